# Initial kernel scaffold; baseline (speedup 1.0000x reference)
import functools

import jax
import jax.numpy as jnp
from jax import lax
from jax.experimental import pallas as pl
from jax.experimental.pallas import tpu as pltpu
from jax.experimental.pallas import tpu_sc as plsc

_NC = 2
_NS = 16
_NW = _NC * _NS
_CH = 80


@functools.lru_cache(maxsize=None)
def _build_spmm(n_nodes, d, e):
    ew_per = e // _NW
    nch = ew_per // _CH
    rpt = n_nodes // _NS
    zr = 125
    nz = rpt // zr
    mesh = plsc.VectorSubcoreMesh(core_axis_name="c", subcore_axis_name="s",
                                  num_cores=_NC)

    @functools.partial(
        pl.kernel,
        mesh=mesh,
        out_type=jax.ShapeDtypeStruct((_NC, n_nodes, d), jnp.float32),
        scratch_types=[
            pltpu.VMEM((nch, _CH), jnp.int32),
            pltpu.VMEM((nch, _CH), jnp.int32),
            pltpu.VMEM((ew_per,), jnp.float32),
            pltpu.VMEM((_CH, d), jnp.float32),
            pltpu.VMEM((zr, d), jnp.float32),
            pltpu.VMEM_SHARED((n_nodes, d), jnp.float32),
            pltpu.SemaphoreType.DMA,
        ],
    )
    def spmm(feat, srcs, dsts, ews, out, src_v, dst_v, ew_v, rows_v, zbuf_v,
             acc, sem):
        cid = lax.axis_index("c")
        sid = lax.axis_index("s")
        wid = cid * _NS + sid

        pltpu.sync_copy(srcs.at[wid], src_v)
        pltpu.sync_copy(dsts.at[wid], dst_v)
        pltpu.sync_copy(ews.at[wid], ew_v)

        def zrow(r, carry):
            for cc in range(d // 16):
                zbuf_v[r, pl.ds(cc * 16, 16)] = jnp.zeros((16,), jnp.float32)
            return carry
        lax.fori_loop(0, zr, zrow, 0)
        for k in range(nz):
            pltpu.sync_copy(zbuf_v, acc.at[pl.ds(sid * rpt + k * zr, zr)])
        plsc.subcore_barrier()

        def chunk(j, carry):
            pltpu.async_copy(feat.at[src_v.at[j]], rows_v, sem).wait()
            for i in range(_CH):
                w = plsc.load_gather(
                    ew_v, [jnp.full((16,), j * _CH + i, jnp.int32)])
                for cc in range(d // 16):
                    sl = pl.ds(cc * 16, 16)
                    rows_v[i, sl] = rows_v[i, sl] * w
            pltpu.sync_copy(rows_v, acc.at[dst_v.at[j]], add=True)
            return carry
        lax.fori_loop(0, nch, chunk, 0)

        plsc.subcore_barrier()
        for k in range(nz):
            sl = pl.ds(sid * rpt + k * zr, zr)
            pltpu.sync_copy(acc.at[sl], zbuf_v)
            pltpu.sync_copy(zbuf_v, out.at[cid, sl])

    return spmm


@functools.lru_cache(maxsize=None)
def _build_finish(n_nodes, d_in, d_out):
    bn = 1000

    def body(p_ref, w_ref, o_ref):
        s = p_ref[0] + p_ref[1]
        o_ref[...] = jnp.dot(s, w_ref[...], preferred_element_type=jnp.float32)

    return pl.pallas_call(
        body,
        grid=(n_nodes // bn,),
        in_specs=[
            pl.BlockSpec((2, bn, d_in), lambda i: (0, i, 0)),
            pl.BlockSpec((d_in, d_out), lambda i: (0, 0)),
        ],
        out_specs=pl.BlockSpec((bn, d_out), lambda i: (i, 0)),
        out_shape=jax.ShapeDtypeStruct((n_nodes, d_out), jnp.float32),
    )


def kernel(features, adj_edge_index, adj_edge_weight, weight):
    n_nodes, d_in = features.shape
    d_out = weight.shape[1]
    e = adj_edge_index.shape[1]
    ew_per = e // _NW
    nch = ew_per // _CH

    src = adj_edge_index[1].reshape(_NW, nch, _CH)
    dst = adj_edge_index[0].reshape(_NW, nch, _CH)
    ews = adj_edge_weight.reshape(_NW, ew_per)

    partials = _build_spmm(n_nodes, d_in, e)(features, src, dst, ews)
    return _build_finish(n_nodes, d_in, d_out)(partials, weight)


# speedup vs baseline: 6.5395x; 6.5395x over previous
import functools

import jax
import jax.numpy as jnp
from jax import lax
from jax.experimental import pallas as pl
from jax.experimental.pallas import tpu as pltpu
from jax.experimental.pallas import tpu_sc as plsc

_NC = 2
_NS = 16
_NW = _NC * _NS
_CH = 80
_GC = 25
_ZR = 32


@functools.lru_cache(maxsize=None)
def _build_spmm(n_nodes, d, e):
    ew_per = e // _NW
    ng = ew_per // (_GC * _CH)
    n_pad = ((n_nodes + 128 * _NS - 1) // (128 * _NS)) * (128 * _NS)
    rpt = n_pad // _NS
    nz = rpt // _ZR
    mesh = plsc.VectorSubcoreMesh(core_axis_name="c", subcore_axis_name="s",
                                  num_cores=_NC)

    @functools.partial(
        pl.kernel,
        mesh=mesh,
        out_type=jax.ShapeDtypeStruct((_NC, n_pad, d), jnp.float32),
        scratch_types=[
            pltpu.VMEM((_GC, _CH), jnp.int32),
            pltpu.VMEM((_GC, _CH), jnp.int32),
            pltpu.VMEM((_GC, _CH), jnp.float32),
            pltpu.VMEM((_CH, d), jnp.float32),
            pltpu.VMEM((_ZR, d), jnp.float32),
            pltpu.VMEM_SHARED((n_pad, d), jnp.float32),
            pltpu.SemaphoreType.DMA,
        ],
    )
    def spmm(feat, srcs, dsts, ews, out, src_v, dst_v, ew_v, rows_v, zbuf_v,
             acc, sem):
        cid = lax.axis_index("c")
        sid = lax.axis_index("s")
        wid = cid * _NS + sid

        for r in range(_ZR):
            for cc in range(d // 16):
                zbuf_v[r, pl.ds(cc * 16, 16)] = jnp.zeros((16,), jnp.float32)
        for k in range(nz):
            pltpu.sync_copy(zbuf_v, acc.at[pl.ds(sid * rpt + k * _ZR, _ZR)])
        plsc.subcore_barrier()

        def group(g, carry):
            blk = wid * ng + g
            pltpu.sync_copy(srcs.at[blk], src_v)
            pltpu.sync_copy(dsts.at[blk], dst_v)
            pltpu.sync_copy(ews.at[blk], ew_v)

            def chunk(j, carry2):
                pltpu.async_copy(feat.at[src_v.at[j]], rows_v, sem).wait()
                for gg in range(_CH // 16):
                    w16 = ew_v[j, pl.ds(gg * 16, 16)]
                    for ee in range(16):
                        i = gg * 16 + ee
                        w = lax.gather(
                            w16, jnp.full((16, 1), ee, jnp.int32),
                            lax.GatherDimensionNumbers(
                                offset_dims=(), collapsed_slice_dims=(0,),
                                start_index_map=(0,)),
                            (1,),
                            mode=lax.GatherScatterMode.PROMISE_IN_BOUNDS)
                        for cc in range(d // 16):
                            sl = pl.ds(cc * 16, 16)
                            rows_v[i, sl] = rows_v[i, sl] * w
                pltpu.sync_copy(rows_v, acc.at[dst_v.at[j]], add=True)
                return carry2
            lax.fori_loop(0, _GC, chunk, 0)
            return carry
        lax.fori_loop(0, ng, group, 0)

        plsc.subcore_barrier()
        for k in range(nz):
            sl = pl.ds(sid * rpt + k * _ZR, _ZR)
            pltpu.sync_copy(acc.at[sl], zbuf_v)
            pltpu.sync_copy(zbuf_v, out.at[cid, sl])

    return spmm


@functools.lru_cache(maxsize=None)
def _build_finish(n_nodes, n_pad, d_in, d_out):
    bn = 1000

    def body(p_ref, w_ref, o_ref):
        s = p_ref[0] + p_ref[1]
        o_ref[...] = jnp.dot(s, w_ref[...], preferred_element_type=jnp.float32)

    return pl.pallas_call(
        body,
        grid=(n_nodes // bn,),
        in_specs=[
            pl.BlockSpec((2, bn, d_in), lambda i: (0, i, 0)),
            pl.BlockSpec((d_in, d_out), lambda i: (0, 0)),
        ],
        out_specs=pl.BlockSpec((bn, d_out), lambda i: (i, 0)),
        out_shape=jax.ShapeDtypeStruct((n_nodes, d_out), jnp.float32),
    )


def kernel(features, adj_edge_index, adj_edge_weight, weight):
    n_nodes, d_in = features.shape
    d_out = weight.shape[1]
    e = adj_edge_index.shape[1]
    nblk = e // (_GC * _CH)
    n_pad = ((n_nodes + 128 * _NS - 1) // (128 * _NS)) * (128 * _NS)

    src = adj_edge_index[1].reshape(nblk, _GC, _CH)
    dst = adj_edge_index[0].reshape(nblk, _GC, _CH)
    ews = adj_edge_weight.reshape(nblk, _GC, _CH)

    partials = _build_spmm(n_nodes, d_in, e)(features, src, dst, ews)
    return _build_finish(n_nodes, n_pad, d_in, d_out)(partials, weight)
